# baseline (speedup 1.0000x reference)
import functools

import jax
import jax.numpy as jnp
from jax import lax
from jax.experimental import pallas as pl
from jax.experimental.pallas import tpu as pltpu
from jax.experimental.pallas import tpu_sc as plsc

N = 10000
E = 320000
D_IN = 128
D_HID = 256

NC = 2
NS = 16
NW = NC * NS
NP = 10240
ROWS_W = NP // NS
C = 128
EPW = E // NW
NFULL = EPW // C
TAIL = EPW - NFULL * C
EPS = E // NS
NFULL2 = EPS // C
TAIL2 = EPS - NFULL2 * C
CS = 64
NFULLS = EPW // CS
TAILS = EPW - NFULLS * CS

f32 = jnp.float32
i32 = jnp.int32

_mesh = plsc.VectorSubcoreMesh(core_axis_name="c", subcore_axis_name="s")


@functools.partial(
    pl.kernel,
    out_type=(jax.ShapeDtypeStruct((NC, NP, D_IN), f32),
              jax.ShapeDtypeStruct((NW, NP), f32)),
    mesh=_mesh,
    scratch_types=(
        pltpu.VMEM((C,), i32), pltpu.VMEM((C,), i32),
        pltpu.VMEM((C,), i32), pltpu.VMEM((C,), i32),
        pltpu.VMEM((TAIL,), i32), pltpu.VMEM((TAIL,), i32),
        pltpu.VMEM((C, D_IN), f32), pltpu.VMEM((C, D_IN), f32),
        pltpu.VMEM((TAIL, D_IN), f32),
        pltpu.VMEM((NP,), f32),
        pltpu.VMEM_SHARED((NP, D_IN), f32),
        pltpu.SemaphoreType.DMA, pltpu.SemaphoreType.DMA,
    ),
    compiler_params=pltpu.CompilerParams(needs_layout_passes=False,
                                         disable_bounds_checks=True),
)
def _sc_agg1(x_hbm, src_hbm, dst_hbm, z_hbm, z1d_hbm,
             agg_out, deg_out,
             srcv0, dstv0, srcv1, dstv1, srcv_t, dstv_t,
             rows0, rows1, rows_t, hist,
             agg_sh, sem0, sem1):
    cid = lax.axis_index("c")
    sid = lax.axis_index("s")
    w = cid * NS + sid
    pltpu.sync_copy(z_hbm, rows0)
    pltpu.sync_copy(z1d_hbm, hist)
    for t in range(ROWS_W // C):
        rr = pl.ds(sid * ROWS_W + t * C, C)
        pltpu.sync_copy(rows0, agg_sh.at[rr])
    plsc.subcore_barrier()
    base0 = w * EPW
    ones16 = jnp.ones((16,), f32)
    srcv = (srcv0, srcv1)
    dstv = (dstv0, dstv1)
    rows = (rows0, rows1)
    sems = (sem0, sem1)

    def count(dref, n):
        for k in range(n // 16):
            idx16 = dref[pl.ds(k * 16, 16)]
            plsc.addupdate_scatter(hist, [idx16], ones16)

    def fire(j, b):
        base = base0 + j * C
        pltpu.sync_copy(src_hbm.at[pl.ds(base, C)], srcv[b])
        pltpu.sync_copy(dst_hbm.at[pl.ds(base, C)], dstv[b])
        pltpu.async_copy(x_hbm.at[srcv[b]], rows[b], sems[b])

    def drain_and_scatter(b):
        pltpu.make_async_copy(x_hbm.at[srcv[b]], rows[b], sems[b]).wait()
        pltpu.sync_copy(rows[b], agg_sh.at[dstv[b]], add=True)
        count(dstv[b], C)

    fire(0, 0)

    def outer(it, _):
        i0 = it * 2
        fire(i0 + 1, 1)
        drain_and_scatter(0)
        pl.when(i0 + 2 < NFULL)(lambda: fire(i0 + 2, 0))
        drain_and_scatter(1)
        return 0

    lax.fori_loop(0, NFULL // 2, outer, 0)
    baset = base0 + NFULL * C
    pltpu.sync_copy(src_hbm.at[pl.ds(baset, TAIL)], srcv_t)
    pltpu.sync_copy(dst_hbm.at[pl.ds(baset, TAIL)], dstv_t)
    pltpu.async_copy(x_hbm.at[srcv_t], rows_t, sem0).wait()
    pltpu.sync_copy(rows_t, agg_sh.at[dstv_t], add=True)
    count(dstv_t, TAIL)
    pltpu.sync_copy(hist, deg_out.at[w])
    plsc.subcore_barrier()
    for t in range(ROWS_W // C):
        rr = pl.ds(sid * ROWS_W + t * C, C)
        pltpu.sync_copy(agg_sh.at[rr], rows0)
        pltpu.sync_copy(rows0, agg_out.at[cid, rr])


@functools.partial(
    pl.kernel,
    out_type=jax.ShapeDtypeStruct((NC, NP, D_IN), f32),
    mesh=_mesh,
    scratch_types=(
        pltpu.VMEM((C,), i32), pltpu.VMEM((C,), i32),
        pltpu.VMEM((C,), i32), pltpu.VMEM((C,), i32),
        pltpu.VMEM((TAIL2,), i32), pltpu.VMEM((TAIL2,), i32),
        pltpu.VMEM((C, D_IN), f32), pltpu.VMEM((C, D_IN), f32),
        pltpu.VMEM((TAIL2, D_IN), f32),
        pltpu.VMEM_SHARED((NP, D_IN), f32),
        pltpu.SemaphoreType.DMA, pltpu.SemaphoreType.DMA,
    ),
    compiler_params=pltpu.CompilerParams(needs_layout_passes=False,
                                         disable_bounds_checks=True),
)
def _sc_agg2(h1a_hbm, h1b_hbm, src_hbm, dst_hbm, z_hbm,
             agg_out,
             srcv0, dstv0, srcv1, dstv1, srcv_t, dstv_t,
             rows0, rows1, rows_t,
             agg_sh, sem0, sem1):
    cid = lax.axis_index("c")
    sid = lax.axis_index("s")
    pltpu.sync_copy(z_hbm, rows0)
    for t in range(ROWS_W // C):
        rr = pl.ds(sid * ROWS_W + t * C, C)
        pltpu.sync_copy(rows0, agg_sh.at[rr])
    plsc.subcore_barrier()
    base0 = sid * EPS
    srcv = (srcv0, srcv1)
    dstv = (dstv0, dstv1)
    rows = (rows0, rows1)
    sems = (sem0, sem1)

    def fire(j, b):
        base = base0 + j * C
        pltpu.sync_copy(src_hbm.at[pl.ds(base, C)], srcv[b])
        pltpu.sync_copy(dst_hbm.at[pl.ds(base, C)], dstv[b])

        @pl.when(cid == 0)
        def _():
            pltpu.async_copy(h1a_hbm.at[srcv[b]], rows[b], sems[b])

        @pl.when(cid == 1)
        def _():
            pltpu.async_copy(h1b_hbm.at[srcv[b]], rows[b], sems[b])

    def drain_and_scatter(b):
        pltpu.make_async_copy(h1a_hbm.at[srcv[b]], rows[b], sems[b]).wait()
        pltpu.sync_copy(rows[b], agg_sh.at[dstv[b]], add=True)

    fire(0, 0)

    def outer(it, _):
        i0 = it * 2
        fire(i0 + 1, 1)
        drain_and_scatter(0)
        pl.when(i0 + 2 < NFULL2)(lambda: fire(i0 + 2, 0))
        drain_and_scatter(1)
        return 0

    lax.fori_loop(0, NFULL2 // 2, outer, 0)
    baset = base0 + NFULL2 * C
    pltpu.sync_copy(src_hbm.at[pl.ds(baset, TAIL2)], srcv_t)
    pltpu.sync_copy(dst_hbm.at[pl.ds(baset, TAIL2)], dstv_t)
    @pl.when(cid == 0)
    def _():
        pltpu.async_copy(h1a_hbm.at[srcv_t], rows_t, sem0)

    @pl.when(cid == 1)
    def _():
        pltpu.async_copy(h1b_hbm.at[srcv_t], rows_t, sem0)

    pltpu.make_async_copy(h1a_hbm.at[srcv_t], rows_t, sem0).wait()
    pltpu.sync_copy(rows_t, agg_sh.at[dstv_t], add=True)
    plsc.subcore_barrier()
    for t in range(ROWS_W // C):
        rr = pl.ds(sid * ROWS_W + t * C, C)
        pltpu.sync_copy(agg_sh.at[rr], rows0)
        pltpu.sync_copy(rows0, agg_out.at[cid, rr])


@functools.partial(
    pl.kernel,
    out_type=jax.ShapeDtypeStruct((E,), f32),
    mesh=_mesh,
    scratch_types=(
        pltpu.VMEM((CS,), i32), pltpu.VMEM((CS,), i32),
        pltpu.VMEM((CS,), i32), pltpu.VMEM((CS,), i32),
        pltpu.VMEM((TAILS,), i32), pltpu.VMEM((TAILS,), i32),
        pltpu.VMEM((CS, D_HID), f32), pltpu.VMEM((CS, D_HID), f32),
        pltpu.VMEM((CS, D_HID), f32), pltpu.VMEM((CS, D_HID), f32),
        pltpu.VMEM((TAILS, D_HID), f32), pltpu.VMEM((TAILS, D_HID), f32),
        pltpu.VMEM((CS,), f32), pltpu.VMEM((CS,), f32),
        pltpu.VMEM((TAILS,), f32),
        pltpu.SemaphoreType.DMA, pltpu.SemaphoreType.DMA,
        pltpu.SemaphoreType.DMA, pltpu.SemaphoreType.DMA,
    ),
    compiler_params=pltpu.CompilerParams(needs_layout_passes=False,
                                         disable_bounds_checks=True),
)
def _sc_score(h2_hbm, src_hbm, dst_hbm,
              score_out,
              srcv0, dstv0, srcv1, dstv1, srcv_t, dstv_t,
              rs0, rd0, rs1, rd1, rs_t, rd_t,
              sc0, sc1, sc_t,
              sems0, semd0, sems1, semd1):
    cid = lax.axis_index("c")
    sid = lax.axis_index("s")
    w = cid * NS + sid
    base0 = w * EPW
    srcv = (srcv0, srcv1)
    dstv = (dstv0, dstv1)
    rs = (rs0, rs1)
    rd = (rd0, rd1)
    sc = (sc0, sc1)
    sems = (sems0, sems1)
    semd = (semd0, semd1)

    zero16 = jnp.zeros((16,), f32)

    def dot_chunk(rs_ref, rd_ref, sc_ref, n_edges):
        for k in range(n_edges // 16):
            sc_ref[pl.ds(k * 16, 16)] = zero16

        def quad(q, _):
            for u in range(4):
                e = q * 4 + u
                a = [zero16, zero16, zero16, zero16]
                for j in range(D_HID // 16):
                    sl = pl.ds(j * 16, 16)
                    a[j % 4] = a[j % 4] + rs_ref[e, sl] * rd_ref[e, sl]
                acc = (a[0] + a[1]) + (a[2] + a[3])
                plsc.addupdate_scatter(sc_ref, [jnp.full((16,), e, dtype=i32)],
                                       acc)
            return 0

        lax.fori_loop(0, n_edges // 4, quad, 0)

    def fire(j, b):
        base = base0 + j * CS
        pltpu.sync_copy(src_hbm.at[pl.ds(base, CS)], srcv[b])
        pltpu.sync_copy(dst_hbm.at[pl.ds(base, CS)], dstv[b])
        pltpu.async_copy(h2_hbm.at[srcv[b]], rs[b], sems[b])
        pltpu.async_copy(h2_hbm.at[dstv[b]], rd[b], semd[b])

    def compute(j, b):
        base = base0 + j * CS
        pltpu.make_async_copy(h2_hbm.at[srcv[b]], rs[b], sems[b]).wait()
        pltpu.make_async_copy(h2_hbm.at[dstv[b]], rd[b], semd[b]).wait()
        dot_chunk(rs[b], rd[b], sc[b], CS)
        pltpu.sync_copy(sc[b], score_out.at[pl.ds(base, CS)])

    fire(0, 0)

    def outer(it, _):
        i0 = it * 2
        fire(i0 + 1, 1)
        compute(i0, 0)
        pl.when(i0 + 2 < NFULLS)(lambda: fire(i0 + 2, 0))
        compute(i0 + 1, 1)
        return 0

    lax.fori_loop(0, NFULLS // 2, outer, 0)
    baset = base0 + NFULLS * CS
    pltpu.sync_copy(src_hbm.at[pl.ds(baset, TAILS)], srcv_t)
    pltpu.sync_copy(dst_hbm.at[pl.ds(baset, TAILS)], dstv_t)
    cp1 = pltpu.async_copy(h2_hbm.at[srcv_t], rs_t, sems0)
    cp2 = pltpu.async_copy(h2_hbm.at[dstv_t], rd_t, semd0)
    cp1.wait()
    cp2.wait()
    dot_chunk(rs_t, rd_t, sc_t, TAILS)
    pltpu.sync_copy(sc_t, score_out.at[pl.ds(baset, TAILS)])


BN = 1024


def _tc1_body(x_ref, a0_ref, a1_ref, d_ref, ws_ref, wn_ref, b_ref,
              ha_ref, hb_ref):
    deg = jnp.sum(d_ref[...], axis=0)[:, None]
    inv = 1.0 / jnp.maximum(deg, 1.0)
    hn = (a0_ref[...] + a1_ref[...]) * inv
    h = (jnp.dot(x_ref[...], ws_ref[...], preferred_element_type=f32)
         + jnp.dot(hn, wn_ref[...], preferred_element_type=f32)
         + b_ref[...])
    h = jnp.maximum(h, 0.0)
    ha_ref[...] = h[:, :D_IN]
    hb_ref[...] = h[:, D_IN:]


_tc1 = pl.pallas_call(
    _tc1_body,
    grid=(NP // BN,),
    in_specs=[
        pl.BlockSpec((BN, D_IN), lambda i: (i, 0)),
        pl.BlockSpec((BN, D_IN), lambda i: (i, 0)),
        pl.BlockSpec((BN, D_IN), lambda i: (i, 0)),
        pl.BlockSpec((NW, BN), lambda i: (0, i)),
        pl.BlockSpec((D_IN, D_HID), lambda i: (0, 0)),
        pl.BlockSpec((D_IN, D_HID), lambda i: (0, 0)),
        pl.BlockSpec((1, D_HID), lambda i: (0, 0)),
    ],
    out_specs=[pl.BlockSpec((BN, D_IN), lambda i: (i, 0)),
               pl.BlockSpec((BN, D_IN), lambda i: (i, 0))],
    out_shape=[jax.ShapeDtypeStruct((NP, D_IN), f32),
               jax.ShapeDtypeStruct((NP, D_IN), f32)],
)


def _tc2_body(ha_ref, hb_ref, a0_ref, a1_ref, d_ref, ws_ref, wn_ref,
              b_ref, h2_ref):
    deg = jnp.sum(d_ref[...], axis=0)[:, None]
    inv = 1.0 / jnp.maximum(deg, 1.0)
    h1 = jnp.concatenate([ha_ref[...], hb_ref[...]], axis=1)
    hn = jnp.concatenate([a0_ref[...], a1_ref[...]], axis=1) * inv
    h2 = (jnp.dot(h1, ws_ref[...], preferred_element_type=f32)
          + jnp.dot(hn, wn_ref[...], preferred_element_type=f32)
          + b_ref[...])
    h2_ref[...] = jnp.maximum(h2, 0.0)


_tc2 = pl.pallas_call(
    _tc2_body,
    grid=(NP // BN,),
    in_specs=[
        pl.BlockSpec((BN, D_IN), lambda i: (i, 0)),
        pl.BlockSpec((BN, D_IN), lambda i: (i, 0)),
        pl.BlockSpec((BN, D_IN), lambda i: (i, 0)),
        pl.BlockSpec((BN, D_IN), lambda i: (i, 0)),
        pl.BlockSpec((NW, BN), lambda i: (0, i)),
        pl.BlockSpec((D_HID, D_HID), lambda i: (0, 0)),
        pl.BlockSpec((D_HID, D_HID), lambda i: (0, 0)),
        pl.BlockSpec((1, D_HID), lambda i: (0, 0)),
    ],
    out_specs=pl.BlockSpec((BN, D_HID), lambda i: (i, 0)),
    out_shape=jax.ShapeDtypeStruct((NP, D_HID), f32),
)


def kernel(x, edge_index, W_self1, W_neigh1, b1, W_self2, W_neigh2, b2):
    src = edge_index[0].astype(i32)
    dst = edge_index[1].astype(i32)
    xp = jnp.pad(x, ((0, NP - N), (0, 0)))
    z = jnp.zeros((C, D_IN), f32)
    z1d = jnp.zeros((NP,), f32)
    aggp, degp = _sc_agg1(xp, src, dst, z, z1d)
    h1a, h1b = _tc1(xp, aggp[0], aggp[1], degp,
                    W_self1, W_neigh1, b1.reshape(1, -1))
    agg2p = _sc_agg2(h1a, h1b, src, dst, z)
    h2p = _tc2(h1a, h1b, agg2p[0], agg2p[1], degp,
               W_self2, W_neigh2, b2.reshape(1, -1))
    score = _sc_score(h2p, src, dst)
    return score.reshape(E, 1)


# speedup vs baseline: 1.0635x; 1.0635x over previous
import functools

import jax
import jax.numpy as jnp
from jax import lax
from jax.experimental import pallas as pl
from jax.experimental.pallas import tpu as pltpu
from jax.experimental.pallas import tpu_sc as plsc

N = 10000
E = 320000
D_IN = 128
D_HID = 256

NC = 2
NS = 16
NW = NC * NS
NP = 10240
ROWS_W = NP // NS
C = 128
EPW = E // NW
NFULL = EPW // C
TAIL = EPW - NFULL * C
EPS = E // NS
NFULL2 = EPS // C
TAIL2 = EPS - NFULL2 * C
CS = 64
NFULLS = EPW // CS
TAILS = EPW - NFULLS * CS

f32 = jnp.float32
i32 = jnp.int32

_mesh = plsc.VectorSubcoreMesh(core_axis_name="c", subcore_axis_name="s")


@functools.partial(
    pl.kernel,
    out_type=(jax.ShapeDtypeStruct((NC, NP, D_IN), f32),
              jax.ShapeDtypeStruct((NW, NP), f32)),
    mesh=_mesh,
    scratch_types=(
        pltpu.VMEM((C,), i32), pltpu.VMEM((C,), i32),
        pltpu.VMEM((C,), i32), pltpu.VMEM((C,), i32),
        pltpu.VMEM((TAIL,), i32), pltpu.VMEM((TAIL,), i32),
        pltpu.VMEM((C, D_IN), f32), pltpu.VMEM((C, D_IN), f32),
        pltpu.VMEM((TAIL, D_IN), f32),
        pltpu.VMEM((NP,), f32),
        pltpu.VMEM_SHARED((NP, D_IN), f32),
        pltpu.SemaphoreType.DMA, pltpu.SemaphoreType.DMA,
    ),
    compiler_params=pltpu.CompilerParams(needs_layout_passes=False,
                                         disable_bounds_checks=True),
)
def _sc_agg1(x_hbm, src_hbm, dst_hbm, z_hbm, z1d_hbm,
             agg_out, deg_out,
             srcv0, dstv0, srcv1, dstv1, srcv_t, dstv_t,
             rows0, rows1, rows_t, hist,
             agg_sh, sem0, sem1):
    cid = lax.axis_index("c")
    sid = lax.axis_index("s")
    w = cid * NS + sid
    pltpu.sync_copy(z_hbm, rows0)
    pltpu.sync_copy(z1d_hbm, hist)
    for t in range(ROWS_W // C):
        rr = pl.ds(sid * ROWS_W + t * C, C)
        pltpu.sync_copy(rows0, agg_sh.at[rr])
    plsc.subcore_barrier()
    base0 = w * EPW
    ones16 = jnp.ones((16,), f32)
    srcv = (srcv0, srcv1)
    dstv = (dstv0, dstv1)
    rows = (rows0, rows1)
    sems = (sem0, sem1)

    def count(dref, n):
        for k in range(n // 16):
            idx16 = dref[pl.ds(k * 16, 16)]
            plsc.addupdate_scatter(hist, [idx16], ones16)

    def fire(j, b):
        base = base0 + j * C
        pltpu.sync_copy(src_hbm.at[pl.ds(base, C)], srcv[b])
        pltpu.sync_copy(dst_hbm.at[pl.ds(base, C)], dstv[b])
        pltpu.async_copy(x_hbm.at[srcv[b]], rows[b], sems[b])

    def drain_and_scatter(b):
        pltpu.make_async_copy(x_hbm.at[srcv[b]], rows[b], sems[b]).wait()
        pltpu.sync_copy(rows[b], agg_sh.at[dstv[b]], add=True)
        count(dstv[b], C)

    fire(0, 0)

    def outer(it, _):
        i0 = it * 2
        fire(i0 + 1, 1)
        drain_and_scatter(0)
        pl.when(i0 + 2 < NFULL)(lambda: fire(i0 + 2, 0))
        drain_and_scatter(1)
        return 0

    lax.fori_loop(0, NFULL // 2, outer, 0)
    baset = base0 + NFULL * C
    pltpu.sync_copy(src_hbm.at[pl.ds(baset, TAIL)], srcv_t)
    pltpu.sync_copy(dst_hbm.at[pl.ds(baset, TAIL)], dstv_t)
    pltpu.async_copy(x_hbm.at[srcv_t], rows_t, sem0).wait()
    pltpu.sync_copy(rows_t, agg_sh.at[dstv_t], add=True)
    count(dstv_t, TAIL)
    pltpu.sync_copy(hist, deg_out.at[w])
    plsc.subcore_barrier()
    for t in range(ROWS_W // C):
        rr = pl.ds(sid * ROWS_W + t * C, C)
        pltpu.sync_copy(agg_sh.at[rr], rows0)
        pltpu.sync_copy(rows0, agg_out.at[cid, rr])


@functools.partial(
    pl.kernel,
    out_type=jax.ShapeDtypeStruct((NC, NP, D_IN), f32),
    mesh=_mesh,
    scratch_types=(
        pltpu.VMEM((C,), i32), pltpu.VMEM((C,), i32),
        pltpu.VMEM((C,), i32), pltpu.VMEM((C,), i32),
        pltpu.VMEM((TAIL2,), i32), pltpu.VMEM((TAIL2,), i32),
        pltpu.VMEM((C, D_IN), f32), pltpu.VMEM((C, D_IN), f32),
        pltpu.VMEM((TAIL2, D_IN), f32),
        pltpu.VMEM_SHARED((NP, D_IN), f32),
        pltpu.SemaphoreType.DMA, pltpu.SemaphoreType.DMA,
    ),
    compiler_params=pltpu.CompilerParams(needs_layout_passes=False,
                                         disable_bounds_checks=True),
)
def _sc_agg2(h1a_hbm, h1b_hbm, src_hbm, dst_hbm, z_hbm,
             agg_out,
             srcv0, dstv0, srcv1, dstv1, srcv_t, dstv_t,
             rows0, rows1, rows_t,
             agg_sh, sem0, sem1):
    cid = lax.axis_index("c")
    sid = lax.axis_index("s")
    pltpu.sync_copy(z_hbm, rows0)
    for t in range(ROWS_W // C):
        rr = pl.ds(sid * ROWS_W + t * C, C)
        pltpu.sync_copy(rows0, agg_sh.at[rr])
    plsc.subcore_barrier()
    base0 = sid * EPS
    srcv = (srcv0, srcv1)
    dstv = (dstv0, dstv1)
    rows = (rows0, rows1)
    sems = (sem0, sem1)

    def fire(j, b):
        base = base0 + j * C
        pltpu.sync_copy(src_hbm.at[pl.ds(base, C)], srcv[b])
        pltpu.sync_copy(dst_hbm.at[pl.ds(base, C)], dstv[b])

        @pl.when(cid == 0)
        def _():
            pltpu.async_copy(h1a_hbm.at[srcv[b]], rows[b], sems[b])

        @pl.when(cid == 1)
        def _():
            pltpu.async_copy(h1b_hbm.at[srcv[b]], rows[b], sems[b])

    def drain_and_scatter(b):
        pltpu.make_async_copy(h1a_hbm.at[srcv[b]], rows[b], sems[b]).wait()
        pltpu.sync_copy(rows[b], agg_sh.at[dstv[b]], add=True)

    fire(0, 0)

    def outer(it, _):
        i0 = it * 2
        fire(i0 + 1, 1)
        drain_and_scatter(0)
        pl.when(i0 + 2 < NFULL2)(lambda: fire(i0 + 2, 0))
        drain_and_scatter(1)
        return 0

    lax.fori_loop(0, NFULL2 // 2, outer, 0)
    baset = base0 + NFULL2 * C
    pltpu.sync_copy(src_hbm.at[pl.ds(baset, TAIL2)], srcv_t)
    pltpu.sync_copy(dst_hbm.at[pl.ds(baset, TAIL2)], dstv_t)
    @pl.when(cid == 0)
    def _():
        pltpu.async_copy(h1a_hbm.at[srcv_t], rows_t, sem0)

    @pl.when(cid == 1)
    def _():
        pltpu.async_copy(h1b_hbm.at[srcv_t], rows_t, sem0)

    pltpu.make_async_copy(h1a_hbm.at[srcv_t], rows_t, sem0).wait()
    pltpu.sync_copy(rows_t, agg_sh.at[dstv_t], add=True)
    plsc.subcore_barrier()
    for t in range(ROWS_W // C):
        rr = pl.ds(sid * ROWS_W + t * C, C)
        pltpu.sync_copy(agg_sh.at[rr], rows0)
        pltpu.sync_copy(rows0, agg_out.at[cid, rr])


@functools.partial(
    pl.kernel,
    out_type=jax.ShapeDtypeStruct((E,), f32),
    mesh=_mesh,
    scratch_types=(
        pltpu.VMEM((CS,), i32), pltpu.VMEM((CS,), i32),
        pltpu.VMEM((CS,), i32), pltpu.VMEM((CS,), i32),
        pltpu.VMEM((TAILS,), i32), pltpu.VMEM((TAILS,), i32),
        pltpu.VMEM((CS, D_HID), f32), pltpu.VMEM((CS, D_HID), f32),
        pltpu.VMEM((CS, D_HID), f32), pltpu.VMEM((CS, D_HID), f32),
        pltpu.VMEM((TAILS, D_HID), f32), pltpu.VMEM((TAILS, D_HID), f32),
        pltpu.VMEM((CS,), f32), pltpu.VMEM((CS,), f32),
        pltpu.VMEM((TAILS,), f32),
        pltpu.SemaphoreType.DMA, pltpu.SemaphoreType.DMA,
        pltpu.SemaphoreType.DMA, pltpu.SemaphoreType.DMA,
    ),
    compiler_params=pltpu.CompilerParams(needs_layout_passes=False,
                                         disable_bounds_checks=True),
)
def _sc_score(h2_hbm, src_hbm, dst_hbm,
              score_out,
              srcv0, dstv0, srcv1, dstv1, srcv_t, dstv_t,
              rs0, rd0, rs1, rd1, rs_t, rd_t,
              sc0, sc1, sc_t,
              sems0, semd0, sems1, semd1):
    cid = lax.axis_index("c")
    sid = lax.axis_index("s")
    w = cid * NS + sid
    base0 = w * EPW
    srcv = (srcv0, srcv1)
    dstv = (dstv0, dstv1)
    rs = (rs0, rs1)
    rd = (rd0, rd1)
    sc = (sc0, sc1)
    sems = (sems0, sems1)
    semd = (semd0, semd1)

    zero16 = jnp.zeros((16,), f32)
    lane0 = lax.iota(i32, 16) == 0

    def dot_chunk(rs_ref, rd_ref, sc_ref, n_edges):
        def quad(q, _):
            for u in range(4):
                e = q * 4 + u
                a = [zero16, zero16, zero16, zero16]
                for j in range(D_HID // 16):
                    sl = pl.ds(j * 16, 16)
                    a[j % 4] = a[j % 4] + rs_ref[e, sl] * rd_ref[e, sl]
                s = jnp.sum((a[0] + a[1]) + (a[2] + a[3]))
                plsc.store_scatter(sc_ref, [jnp.full((16,), e, dtype=i32)],
                                   jnp.full((16,), s, dtype=f32), mask=lane0)
            return 0

        lax.fori_loop(0, n_edges // 4, quad, 0)

    def fire(j, b):
        base = base0 + j * CS
        pltpu.sync_copy(src_hbm.at[pl.ds(base, CS)], srcv[b])
        pltpu.sync_copy(dst_hbm.at[pl.ds(base, CS)], dstv[b])
        pltpu.async_copy(h2_hbm.at[srcv[b]], rs[b], sems[b])
        pltpu.async_copy(h2_hbm.at[dstv[b]], rd[b], semd[b])

    def compute(j, b):
        base = base0 + j * CS
        pltpu.make_async_copy(h2_hbm.at[srcv[b]], rs[b], sems[b]).wait()
        pltpu.make_async_copy(h2_hbm.at[dstv[b]], rd[b], semd[b]).wait()
        dot_chunk(rs[b], rd[b], sc[b], CS)
        pltpu.sync_copy(sc[b], score_out.at[pl.ds(base, CS)])

    fire(0, 0)

    def outer(it, _):
        i0 = it * 2
        fire(i0 + 1, 1)
        compute(i0, 0)
        pl.when(i0 + 2 < NFULLS)(lambda: fire(i0 + 2, 0))
        compute(i0 + 1, 1)
        return 0

    lax.fori_loop(0, NFULLS // 2, outer, 0)
    baset = base0 + NFULLS * CS
    pltpu.sync_copy(src_hbm.at[pl.ds(baset, TAILS)], srcv_t)
    pltpu.sync_copy(dst_hbm.at[pl.ds(baset, TAILS)], dstv_t)
    cp1 = pltpu.async_copy(h2_hbm.at[srcv_t], rs_t, sems0)
    cp2 = pltpu.async_copy(h2_hbm.at[dstv_t], rd_t, semd0)
    cp1.wait()
    cp2.wait()
    dot_chunk(rs_t, rd_t, sc_t, TAILS)
    pltpu.sync_copy(sc_t, score_out.at[pl.ds(baset, TAILS)])


BN = 1024


def _tc1_body(x_ref, a0_ref, a1_ref, d_ref, ws_ref, wn_ref, b_ref,
              ha_ref, hb_ref):
    deg = jnp.sum(d_ref[...], axis=0)[:, None]
    inv = 1.0 / jnp.maximum(deg, 1.0)
    hn = (a0_ref[...] + a1_ref[...]) * inv
    h = (jnp.dot(x_ref[...], ws_ref[...], preferred_element_type=f32)
         + jnp.dot(hn, wn_ref[...], preferred_element_type=f32)
         + b_ref[...])
    h = jnp.maximum(h, 0.0)
    ha_ref[...] = h[:, :D_IN]
    hb_ref[...] = h[:, D_IN:]


_tc1 = pl.pallas_call(
    _tc1_body,
    grid=(NP // BN,),
    in_specs=[
        pl.BlockSpec((BN, D_IN), lambda i: (i, 0)),
        pl.BlockSpec((BN, D_IN), lambda i: (i, 0)),
        pl.BlockSpec((BN, D_IN), lambda i: (i, 0)),
        pl.BlockSpec((NW, BN), lambda i: (0, i)),
        pl.BlockSpec((D_IN, D_HID), lambda i: (0, 0)),
        pl.BlockSpec((D_IN, D_HID), lambda i: (0, 0)),
        pl.BlockSpec((1, D_HID), lambda i: (0, 0)),
    ],
    out_specs=[pl.BlockSpec((BN, D_IN), lambda i: (i, 0)),
               pl.BlockSpec((BN, D_IN), lambda i: (i, 0))],
    out_shape=[jax.ShapeDtypeStruct((NP, D_IN), f32),
               jax.ShapeDtypeStruct((NP, D_IN), f32)],
)


def _tc2_body(ha_ref, hb_ref, a0_ref, a1_ref, d_ref, ws_ref, wn_ref,
              b_ref, h2_ref):
    deg = jnp.sum(d_ref[...], axis=0)[:, None]
    inv = 1.0 / jnp.maximum(deg, 1.0)
    h1 = jnp.concatenate([ha_ref[...], hb_ref[...]], axis=1)
    hn = jnp.concatenate([a0_ref[...], a1_ref[...]], axis=1) * inv
    h2 = (jnp.dot(h1, ws_ref[...], preferred_element_type=f32)
          + jnp.dot(hn, wn_ref[...], preferred_element_type=f32)
          + b_ref[...])
    h2_ref[...] = jnp.maximum(h2, 0.0)


_tc2 = pl.pallas_call(
    _tc2_body,
    grid=(NP // BN,),
    in_specs=[
        pl.BlockSpec((BN, D_IN), lambda i: (i, 0)),
        pl.BlockSpec((BN, D_IN), lambda i: (i, 0)),
        pl.BlockSpec((BN, D_IN), lambda i: (i, 0)),
        pl.BlockSpec((BN, D_IN), lambda i: (i, 0)),
        pl.BlockSpec((NW, BN), lambda i: (0, i)),
        pl.BlockSpec((D_HID, D_HID), lambda i: (0, 0)),
        pl.BlockSpec((D_HID, D_HID), lambda i: (0, 0)),
        pl.BlockSpec((1, D_HID), lambda i: (0, 0)),
    ],
    out_specs=pl.BlockSpec((BN, D_HID), lambda i: (i, 0)),
    out_shape=jax.ShapeDtypeStruct((NP, D_HID), f32),
)


def kernel(x, edge_index, W_self1, W_neigh1, b1, W_self2, W_neigh2, b2):
    src = edge_index[0].astype(i32)
    dst = edge_index[1].astype(i32)
    xp = jnp.pad(x, ((0, NP - N), (0, 0)))
    z = jnp.zeros((C, D_IN), f32)
    z1d = jnp.zeros((NP,), f32)
    aggp, degp = _sc_agg1(xp, src, dst, z, z1d)
    h1a, h1b = _tc1(xp, aggp[0], aggp[1], degp,
                    W_self1, W_neigh1, b1.reshape(1, -1))
    agg2p = _sc_agg2(h1a, h1b, src, dst, z)
    h2p = _tc2(h1a, h1b, agg2p[0], agg2p[1], degp,
               W_self2, W_neigh2, b2.reshape(1, -1))
    score = _sc_score(h2p, src, dst)
    return score.reshape(E, 1)


# speedup vs baseline: 1.0771x; 1.0128x over previous
import functools

import jax
import jax.numpy as jnp
from jax import lax
from jax.experimental import pallas as pl
from jax.experimental.pallas import tpu as pltpu
from jax.experimental.pallas import tpu_sc as plsc

N = 10000
E = 320000
D_IN = 128
D_HID = 256

NC = 2
NS = 16
NW = NC * NS
NP = 10240
ROWS_W = NP // NS
C = 128
EPW = E // NW
NFULL = EPW // C
TAIL = EPW - NFULL * C
EPS = E // NS
NFULL2 = EPS // C
TAIL2 = EPS - NFULL2 * C
CS = 64
NFULLS = EPW // CS
TAILS = EPW - NFULLS * CS

f32 = jnp.float32
i32 = jnp.int32

_mesh = plsc.VectorSubcoreMesh(core_axis_name="c", subcore_axis_name="s")


@functools.partial(
    pl.kernel,
    out_type=(jax.ShapeDtypeStruct((NC, NP, D_IN), f32),
              jax.ShapeDtypeStruct((NW, NP), f32)),
    mesh=_mesh,
    scratch_types=(
        pltpu.VMEM((C,), i32), pltpu.VMEM((C,), i32),
        pltpu.VMEM((C,), i32), pltpu.VMEM((C,), i32),
        pltpu.VMEM((TAIL,), i32), pltpu.VMEM((TAIL,), i32),
        pltpu.VMEM((C, D_IN), f32), pltpu.VMEM((C, D_IN), f32),
        pltpu.VMEM((TAIL, D_IN), f32),
        pltpu.VMEM((NP,), f32),
        pltpu.VMEM_SHARED((NP, D_IN), f32),
        pltpu.SemaphoreType.DMA, pltpu.SemaphoreType.DMA,
    ),
    compiler_params=pltpu.CompilerParams(needs_layout_passes=False,
                                         disable_bounds_checks=True),
)
def _sc_agg1(x_hbm, src_hbm, dst_hbm, z_hbm, z1d_hbm,
             agg_out, deg_out,
             srcv0, dstv0, srcv1, dstv1, srcv_t, dstv_t,
             rows0, rows1, rows_t, hist,
             agg_sh, sem0, sem1):
    cid = lax.axis_index("c")
    sid = lax.axis_index("s")
    w = cid * NS + sid
    pltpu.sync_copy(z_hbm, rows0)
    pltpu.sync_copy(z1d_hbm, hist)
    for t in range(ROWS_W // C):
        rr = pl.ds(sid * ROWS_W + t * C, C)
        pltpu.sync_copy(rows0, agg_sh.at[rr])
    plsc.subcore_barrier()
    base0 = w * EPW
    ones16 = jnp.ones((16,), f32)
    srcv = (srcv0, srcv1)
    dstv = (dstv0, dstv1)
    rows = (rows0, rows1)
    sems = (sem0, sem1)

    def count(dref, n):
        for k in range(n // 16):
            idx16 = dref[pl.ds(k * 16, 16)]
            plsc.addupdate_scatter(hist, [idx16], ones16)

    def fire(j, b):
        base = base0 + j * C
        pltpu.sync_copy(src_hbm.at[pl.ds(base, C)], srcv[b])
        pltpu.sync_copy(dst_hbm.at[pl.ds(base, C)], dstv[b])
        pltpu.async_copy(x_hbm.at[srcv[b]], rows[b], sems[b])

    def drain_and_scatter(b):
        pltpu.make_async_copy(x_hbm.at[srcv[b]], rows[b], sems[b]).wait()
        pltpu.sync_copy(rows[b], agg_sh.at[dstv[b]], add=True)
        count(dstv[b], C)

    fire(0, 0)

    def outer(it, _):
        i0 = it * 2
        fire(i0 + 1, 1)
        drain_and_scatter(0)
        pl.when(i0 + 2 < NFULL)(lambda: fire(i0 + 2, 0))
        drain_and_scatter(1)
        return 0

    lax.fori_loop(0, NFULL // 2, outer, 0)
    baset = base0 + NFULL * C
    pltpu.sync_copy(src_hbm.at[pl.ds(baset, TAIL)], srcv_t)
    pltpu.sync_copy(dst_hbm.at[pl.ds(baset, TAIL)], dstv_t)
    pltpu.async_copy(x_hbm.at[srcv_t], rows_t, sem0).wait()
    pltpu.sync_copy(rows_t, agg_sh.at[dstv_t], add=True)
    count(dstv_t, TAIL)
    pltpu.sync_copy(hist, deg_out.at[w])
    plsc.subcore_barrier()
    for t in range(ROWS_W // C):
        rr = pl.ds(sid * ROWS_W + t * C, C)
        pltpu.sync_copy(agg_sh.at[rr], rows0)
        pltpu.sync_copy(rows0, agg_out.at[cid, rr])


@functools.partial(
    pl.kernel,
    out_type=jax.ShapeDtypeStruct((NC, NP, D_IN), f32),
    mesh=_mesh,
    scratch_types=(
        pltpu.VMEM((C,), i32), pltpu.VMEM((C,), i32),
        pltpu.VMEM((C,), i32), pltpu.VMEM((C,), i32),
        pltpu.VMEM((TAIL2,), i32), pltpu.VMEM((TAIL2,), i32),
        pltpu.VMEM((C, D_IN), f32), pltpu.VMEM((C, D_IN), f32),
        pltpu.VMEM((TAIL2, D_IN), f32),
        pltpu.VMEM_SHARED((NP, D_IN), f32),
        pltpu.SemaphoreType.DMA, pltpu.SemaphoreType.DMA,
    ),
    compiler_params=pltpu.CompilerParams(needs_layout_passes=False,
                                         disable_bounds_checks=True),
)
def _sc_agg2(h1a_hbm, h1b_hbm, src_hbm, dst_hbm, z_hbm,
             agg_out,
             srcv0, dstv0, srcv1, dstv1, srcv_t, dstv_t,
             rows0, rows1, rows_t,
             agg_sh, sem0, sem1):
    cid = lax.axis_index("c")
    sid = lax.axis_index("s")
    pltpu.sync_copy(z_hbm, rows0)
    for t in range(ROWS_W // C):
        rr = pl.ds(sid * ROWS_W + t * C, C)
        pltpu.sync_copy(rows0, agg_sh.at[rr])
    plsc.subcore_barrier()
    base0 = sid * EPS
    srcv = (srcv0, srcv1)
    dstv = (dstv0, dstv1)
    rows = (rows0, rows1)
    sems = (sem0, sem1)

    def fire(j, b):
        base = base0 + j * C
        pltpu.sync_copy(src_hbm.at[pl.ds(base, C)], srcv[b])
        pltpu.sync_copy(dst_hbm.at[pl.ds(base, C)], dstv[b])

        @pl.when(cid == 0)
        def _():
            pltpu.async_copy(h1a_hbm.at[srcv[b]], rows[b], sems[b])

        @pl.when(cid == 1)
        def _():
            pltpu.async_copy(h1b_hbm.at[srcv[b]], rows[b], sems[b])

    def drain_and_scatter(b):
        pltpu.make_async_copy(h1a_hbm.at[srcv[b]], rows[b], sems[b]).wait()
        pltpu.sync_copy(rows[b], agg_sh.at[dstv[b]], add=True)

    fire(0, 0)

    def outer(it, _):
        i0 = it * 2
        fire(i0 + 1, 1)
        drain_and_scatter(0)
        pl.when(i0 + 2 < NFULL2)(lambda: fire(i0 + 2, 0))
        drain_and_scatter(1)
        return 0

    lax.fori_loop(0, NFULL2 // 2, outer, 0)
    baset = base0 + NFULL2 * C
    pltpu.sync_copy(src_hbm.at[pl.ds(baset, TAIL2)], srcv_t)
    pltpu.sync_copy(dst_hbm.at[pl.ds(baset, TAIL2)], dstv_t)
    @pl.when(cid == 0)
    def _():
        pltpu.async_copy(h1a_hbm.at[srcv_t], rows_t, sem0)

    @pl.when(cid == 1)
    def _():
        pltpu.async_copy(h1b_hbm.at[srcv_t], rows_t, sem0)

    pltpu.make_async_copy(h1a_hbm.at[srcv_t], rows_t, sem0).wait()
    pltpu.sync_copy(rows_t, agg_sh.at[dstv_t], add=True)
    plsc.subcore_barrier()
    for t in range(ROWS_W // C):
        rr = pl.ds(sid * ROWS_W + t * C, C)
        pltpu.sync_copy(agg_sh.at[rr], rows0)
        pltpu.sync_copy(rows0, agg_out.at[cid, rr])


@functools.partial(
    pl.kernel,
    out_type=jax.ShapeDtypeStruct((E,), f32),
    mesh=_mesh,
    scratch_types=(
        pltpu.VMEM((CS,), i32), pltpu.VMEM((CS,), i32),
        pltpu.VMEM((CS,), i32), pltpu.VMEM((CS,), i32),
        pltpu.VMEM((TAILS,), i32), pltpu.VMEM((TAILS,), i32),
        pltpu.VMEM((CS, 128), i32), pltpu.VMEM((CS, 128), i32),
        pltpu.VMEM((CS, 128), i32), pltpu.VMEM((CS, 128), i32),
        pltpu.VMEM((TAILS, 128), i32), pltpu.VMEM((TAILS, 128), i32),
        pltpu.VMEM((CS,), f32), pltpu.VMEM((CS,), f32),
        pltpu.VMEM((TAILS,), f32),
        pltpu.SemaphoreType.DMA, pltpu.SemaphoreType.DMA,
        pltpu.SemaphoreType.DMA, pltpu.SemaphoreType.DMA,
    ),
    compiler_params=pltpu.CompilerParams(needs_layout_passes=False,
                                         disable_bounds_checks=True),
)
def _sc_score(h2_hbm, src_hbm, dst_hbm,
              score_out,
              srcv0, dstv0, srcv1, dstv1, srcv_t, dstv_t,
              rs0, rd0, rs1, rd1, rs_t, rd_t,
              sc0, sc1, sc_t,
              sems0, semd0, sems1, semd1):
    cid = lax.axis_index("c")
    sid = lax.axis_index("s")
    w = cid * NS + sid
    base0 = w * EPW
    srcv = (srcv0, srcv1)
    dstv = (dstv0, dstv1)
    rs = (rs0, rs1)
    rd = (rd0, rd1)
    sc = (sc0, sc1)
    sems = (sems0, sems1)
    semd = (semd0, semd1)

    zero16 = jnp.zeros((16,), f32)
    lane0 = lax.iota(i32, 16) == 0

    def dot_chunk(rs_ref, rd_ref, sc_ref, n_edges):
        def quad(q, _):
            for u in range(4):
                e = q * 4 + u
                a = [zero16, zero16, zero16, zero16]
                for o in range(8):
                    sl = pl.ds(o * 16, 16)
                    vs = plsc.bitcast(rs_ref[e, sl], jnp.bfloat16)
                    vd = plsc.bitcast(rd_ref[e, sl], jnp.bfloat16)
                    sa, sb = plsc.unpack(
                        vs, format=plsc.PackFormat.INTERLEAVED,
                        preferred_element_type=f32)
                    da, db = plsc.unpack(
                        vd, format=plsc.PackFormat.INTERLEAVED,
                        preferred_element_type=f32)
                    a[o % 4] = a[o % 4] + sa * da
                    a[(o + 2) % 4] = a[(o + 2) % 4] + sb * db
                s = jnp.sum((a[0] + a[1]) + (a[2] + a[3]))
                plsc.store_scatter(sc_ref, [jnp.full((16,), e, dtype=i32)],
                                   jnp.full((16,), s, dtype=f32), mask=lane0)
            return 0

        lax.fori_loop(0, n_edges // 4, quad, 0)

    def fire(j, b):
        base = base0 + j * CS
        pltpu.sync_copy(src_hbm.at[pl.ds(base, CS)], srcv[b])
        pltpu.sync_copy(dst_hbm.at[pl.ds(base, CS)], dstv[b])
        pltpu.async_copy(h2_hbm.at[srcv[b]], rs[b], sems[b])
        pltpu.async_copy(h2_hbm.at[dstv[b]], rd[b], semd[b])

    def compute(j, b):
        base = base0 + j * CS
        pltpu.make_async_copy(h2_hbm.at[srcv[b]], rs[b], sems[b]).wait()
        pltpu.make_async_copy(h2_hbm.at[dstv[b]], rd[b], semd[b]).wait()
        dot_chunk(rs[b], rd[b], sc[b], CS)
        pltpu.sync_copy(sc[b], score_out.at[pl.ds(base, CS)])

    fire(0, 0)

    def outer(it, _):
        i0 = it * 2
        fire(i0 + 1, 1)
        compute(i0, 0)
        pl.when(i0 + 2 < NFULLS)(lambda: fire(i0 + 2, 0))
        compute(i0 + 1, 1)
        return 0

    lax.fori_loop(0, NFULLS // 2, outer, 0)
    baset = base0 + NFULLS * CS
    pltpu.sync_copy(src_hbm.at[pl.ds(baset, TAILS)], srcv_t)
    pltpu.sync_copy(dst_hbm.at[pl.ds(baset, TAILS)], dstv_t)
    cp1 = pltpu.async_copy(h2_hbm.at[srcv_t], rs_t, sems0)
    cp2 = pltpu.async_copy(h2_hbm.at[dstv_t], rd_t, semd0)
    cp1.wait()
    cp2.wait()
    dot_chunk(rs_t, rd_t, sc_t, TAILS)
    pltpu.sync_copy(sc_t, score_out.at[pl.ds(baset, TAILS)])


BN = 1024


def _tc1_body(x_ref, a0_ref, a1_ref, d_ref, ws_ref, wn_ref, b_ref,
              ha_ref, hb_ref):
    deg = jnp.sum(d_ref[...], axis=0)[:, None]
    inv = 1.0 / jnp.maximum(deg, 1.0)
    hn = (a0_ref[...] + a1_ref[...]) * inv
    h = (jnp.dot(x_ref[...], ws_ref[...], preferred_element_type=f32)
         + jnp.dot(hn, wn_ref[...], preferred_element_type=f32)
         + b_ref[...])
    h = jnp.maximum(h, 0.0)
    ha_ref[...] = h[:, :D_IN]
    hb_ref[...] = h[:, D_IN:]


_tc1 = pl.pallas_call(
    _tc1_body,
    grid=(NP // BN,),
    in_specs=[
        pl.BlockSpec((BN, D_IN), lambda i: (i, 0)),
        pl.BlockSpec((BN, D_IN), lambda i: (i, 0)),
        pl.BlockSpec((BN, D_IN), lambda i: (i, 0)),
        pl.BlockSpec((NW, BN), lambda i: (0, i)),
        pl.BlockSpec((D_IN, D_HID), lambda i: (0, 0)),
        pl.BlockSpec((D_IN, D_HID), lambda i: (0, 0)),
        pl.BlockSpec((1, D_HID), lambda i: (0, 0)),
    ],
    out_specs=[pl.BlockSpec((BN, D_IN), lambda i: (i, 0)),
               pl.BlockSpec((BN, D_IN), lambda i: (i, 0))],
    out_shape=[jax.ShapeDtypeStruct((NP, D_IN), f32),
               jax.ShapeDtypeStruct((NP, D_IN), f32)],
)


def _tc2_body(ha_ref, hb_ref, a0_ref, a1_ref, d_ref, ws_ref, wn_ref,
              b_ref, h2_ref):
    deg = jnp.sum(d_ref[...], axis=0)[:, None]
    inv = 1.0 / jnp.maximum(deg, 1.0)
    h1 = jnp.concatenate([ha_ref[...], hb_ref[...]], axis=1)
    hn = jnp.concatenate([a0_ref[...], a1_ref[...]], axis=1) * inv
    h2 = (jnp.dot(h1, ws_ref[...], preferred_element_type=f32)
          + jnp.dot(hn, wn_ref[...], preferred_element_type=f32)
          + b_ref[...])
    h2_ref[...] = jnp.maximum(h2, 0.0).astype(jnp.bfloat16)


_tc2 = pl.pallas_call(
    _tc2_body,
    grid=(NP // BN,),
    in_specs=[
        pl.BlockSpec((BN, D_IN), lambda i: (i, 0)),
        pl.BlockSpec((BN, D_IN), lambda i: (i, 0)),
        pl.BlockSpec((BN, D_IN), lambda i: (i, 0)),
        pl.BlockSpec((BN, D_IN), lambda i: (i, 0)),
        pl.BlockSpec((NW, BN), lambda i: (0, i)),
        pl.BlockSpec((D_HID, D_HID), lambda i: (0, 0)),
        pl.BlockSpec((D_HID, D_HID), lambda i: (0, 0)),
        pl.BlockSpec((1, D_HID), lambda i: (0, 0)),
    ],
    out_specs=pl.BlockSpec((BN, D_HID), lambda i: (i, 0)),
    out_shape=jax.ShapeDtypeStruct((NP, D_HID), jnp.bfloat16),
)


def kernel(x, edge_index, W_self1, W_neigh1, b1, W_self2, W_neigh2, b2):
    src = edge_index[0].astype(i32)
    dst = edge_index[1].astype(i32)
    xp = jnp.pad(x, ((0, NP - N), (0, 0)))
    z = jnp.zeros((C, D_IN), f32)
    z1d = jnp.zeros((NP,), f32)
    aggp, degp = _sc_agg1(xp, src, dst, z, z1d)
    h1a, h1b = _tc1(xp, aggp[0], aggp[1], degp,
                    W_self1, W_neigh1, b1.reshape(1, -1))
    agg2p = _sc_agg2(h1a, h1b, src, dst, z)
    h2p = _tc2(h1a, h1b, agg2p[0], agg2p[1], degp,
               W_self2, W_neigh2, b2.reshape(1, -1))
    h2w = lax.bitcast_convert_type(h2p.reshape(NP, 128, 2), i32)
    score = _sc_score(h2w, src, dst)
    return score.reshape(E, 1)


# speedup vs baseline: 1.3493x; 1.2527x over previous
import functools

import jax
import jax.numpy as jnp
from jax import lax
from jax.experimental import pallas as pl
from jax.experimental.pallas import tpu as pltpu
from jax.experimental.pallas import tpu_sc as plsc

N = 10000
E = 320000
D_IN = 128
D_HID = 256

NC = 2
NS = 16
NW = NC * NS
NP = 10240
ROWS_W = NP // NS
C = 128
EPW = E // NW
NFULL = EPW // C
TAIL = EPW - NFULL * C
EPS = E // NS
NFULL2 = EPS // C
TAIL2 = EPS - NFULL2 * C
CA = 64
NFA = EPW // CA
TAILA = EPW - NFA * CA
NFC = EPS // CA
TAILC = EPS - NFC * CA
CS = 64
NFULLS = EPW // CS
TAILS = EPW - NFULLS * CS

f32 = jnp.float32
i32 = jnp.int32

_mesh = plsc.VectorSubcoreMesh(core_axis_name="c", subcore_axis_name="s")


@functools.partial(
    pl.kernel,
    out_type=(jax.ShapeDtypeStruct((NC, NP, D_IN), f32),
              jax.ShapeDtypeStruct((NW, NP), f32)),
    mesh=_mesh,
    scratch_types=(
        pltpu.VMEM((EPW,), i32),
        pltpu.VMEM((CA,), i32), pltpu.VMEM((CA,), i32),
        pltpu.VMEM((TAILA,), i32), pltpu.VMEM((TAILA,), i32),
        pltpu.VMEM((CA, D_IN), f32), pltpu.VMEM((CA, D_IN), f32),
        pltpu.VMEM((TAILA, D_IN), f32),
        pltpu.VMEM((NP,), f32),
        pltpu.VMEM_SHARED((NP, D_IN), f32),
        pltpu.SemaphoreType.DMA, pltpu.SemaphoreType.DMA,
        pltpu.SemaphoreType.DMA, pltpu.SemaphoreType.DMA,
    ),
    compiler_params=pltpu.CompilerParams(needs_layout_passes=False,
                                         disable_bounds_checks=True),
)
def _sc_agg1(x_hbm, src_hbm, dst_hbm, z_hbm, z1d_hbm,
             agg_out, deg_out,
             srcall, dstv0, dstv1, srcv_t, dstv_t,
             rows0, rows1, rows_t, hist,
             agg_sh, sem0, sem1, semi0, semi1):
    cid = lax.axis_index("c")
    sid = lax.axis_index("s")
    w = cid * NS + sid
    base0 = w * EPW
    pltpu.sync_copy(src_hbm.at[pl.ds(base0, EPW)], srcall)
    pltpu.sync_copy(z_hbm, rows0)
    pltpu.sync_copy(z1d_hbm, hist)
    for t in range(ROWS_W // CA):
        rr = pl.ds(sid * ROWS_W + t * CA, CA)
        pltpu.sync_copy(rows0, agg_sh.at[rr])
    plsc.subcore_barrier()
    ones16 = jnp.ones((16,), f32)
    dstv = (dstv0, dstv1)
    rows = (rows0, rows1)
    sems = (sem0, sem1)
    semi = (semi0, semi1)

    def count(dref, n):
        for k in range(n // 16):
            idx16 = dref[pl.ds(k * 16, 16)]
            plsc.addupdate_scatter(hist, [idx16], ones16)

    def fire(j, b):
        pltpu.async_copy(dst_hbm.at[pl.ds(base0 + j * CA, CA)], dstv[b],
                         semi[b])
        pltpu.async_copy(x_hbm.at[srcall.at[pl.ds(j * CA, CA)]], rows[b],
                         sems[b])

    def drain_and_scatter(j, b):
        pltpu.make_async_copy(dst_hbm.at[pl.ds(base0 + j * CA, CA)], dstv[b],
                              semi[b]).wait()
        pltpu.make_async_copy(x_hbm.at[srcall.at[pl.ds(j * CA, CA)]], rows[b],
                              sems[b]).wait()
        pltpu.sync_copy(rows[b], agg_sh.at[dstv[b]], add=True)
        count(dstv[b], CA)

    fire(0, 0)

    def outer(it, _):
        i0 = it * 2
        fire(i0 + 1, 1)
        drain_and_scatter(i0, 0)
        pl.when(i0 + 2 < NFA)(lambda: fire(i0 + 2, 0))
        drain_and_scatter(i0 + 1, 1)
        return 0

    lax.fori_loop(0, NFA // 2, outer, 0)
    baset = base0 + NFA * CA
    pltpu.sync_copy(src_hbm.at[pl.ds(baset, TAILA)], srcv_t)
    pltpu.sync_copy(dst_hbm.at[pl.ds(baset, TAILA)], dstv_t)
    pltpu.async_copy(x_hbm.at[srcv_t], rows_t, sem0).wait()
    pltpu.sync_copy(rows_t, agg_sh.at[dstv_t], add=True)
    count(dstv_t, TAILA)
    pltpu.sync_copy(hist, deg_out.at[w])
    plsc.subcore_barrier()
    for t in range(ROWS_W // CA):
        rr = pl.ds(sid * ROWS_W + t * CA, CA)
        pltpu.sync_copy(agg_sh.at[rr], rows0)
        pltpu.sync_copy(rows0, agg_out.at[cid, rr])


@functools.partial(
    pl.kernel,
    out_type=jax.ShapeDtypeStruct((NC, NP, D_IN), f32),
    mesh=_mesh,
    scratch_types=(
        pltpu.VMEM((EPS,), i32),
        pltpu.VMEM((CA,), i32), pltpu.VMEM((CA,), i32),
        pltpu.VMEM((TAILC,), i32), pltpu.VMEM((TAILC,), i32),
        pltpu.VMEM((CA, D_IN), f32), pltpu.VMEM((CA, D_IN), f32),
        pltpu.VMEM((TAILC, D_IN), f32),
        pltpu.VMEM_SHARED((NP, D_IN), f32),
        pltpu.SemaphoreType.DMA, pltpu.SemaphoreType.DMA,
        pltpu.SemaphoreType.DMA, pltpu.SemaphoreType.DMA,
    ),
    compiler_params=pltpu.CompilerParams(needs_layout_passes=False,
                                         disable_bounds_checks=True),
)
def _sc_agg2(h1a_hbm, h1b_hbm, src_hbm, dst_hbm, z_hbm,
             agg_out,
             srcall, dstv0, dstv1, srcv_t, dstv_t,
             rows0, rows1, rows_t,
             agg_sh, sem0, sem1, semi0, semi1):
    cid = lax.axis_index("c")
    sid = lax.axis_index("s")
    base0 = sid * EPS
    pltpu.sync_copy(src_hbm.at[pl.ds(base0, EPS)], srcall)
    pltpu.sync_copy(z_hbm, rows0)
    for t in range(ROWS_W // CA):
        rr = pl.ds(sid * ROWS_W + t * CA, CA)
        pltpu.sync_copy(rows0, agg_sh.at[rr])
    plsc.subcore_barrier()
    dstv = (dstv0, dstv1)
    rows = (rows0, rows1)
    sems = (sem0, sem1)
    semi = (semi0, semi1)

    def fire(j, b):
        pltpu.async_copy(dst_hbm.at[pl.ds(base0 + j * CA, CA)], dstv[b],
                         semi[b])

        @pl.when(cid == 0)
        def _():
            pltpu.async_copy(h1a_hbm.at[srcall.at[pl.ds(j * CA, CA)]],
                             rows[b], sems[b])

        @pl.when(cid == 1)
        def _():
            pltpu.async_copy(h1b_hbm.at[srcall.at[pl.ds(j * CA, CA)]],
                             rows[b], sems[b])

    def drain_and_scatter(j, b):
        pltpu.make_async_copy(dst_hbm.at[pl.ds(base0 + j * CA, CA)], dstv[b],
                              semi[b]).wait()
        pltpu.make_async_copy(h1a_hbm.at[srcall.at[pl.ds(j * CA, CA)]],
                              rows[b], sems[b]).wait()
        pltpu.sync_copy(rows[b], agg_sh.at[dstv[b]], add=True)

    fire(0, 0)

    def outer(it, _):
        i0 = it * 2
        fire(i0 + 1, 1)
        drain_and_scatter(i0, 0)
        pl.when(i0 + 2 < NFC)(lambda: fire(i0 + 2, 0))
        drain_and_scatter(i0 + 1, 1)
        return 0

    lax.fori_loop(0, NFC // 2, outer, 0)
    baset = base0 + NFC * CA
    pltpu.sync_copy(src_hbm.at[pl.ds(baset, TAILC)], srcv_t)
    pltpu.sync_copy(dst_hbm.at[pl.ds(baset, TAILC)], dstv_t)
    @pl.when(cid == 0)
    def _():
        pltpu.async_copy(h1a_hbm.at[srcv_t], rows_t, sem0)

    @pl.when(cid == 1)
    def _():
        pltpu.async_copy(h1b_hbm.at[srcv_t], rows_t, sem0)

    pltpu.make_async_copy(h1a_hbm.at[srcv_t], rows_t, sem0).wait()
    pltpu.sync_copy(rows_t, agg_sh.at[dstv_t], add=True)
    plsc.subcore_barrier()
    for t in range(ROWS_W // CA):
        rr = pl.ds(sid * ROWS_W + t * CA, CA)
        pltpu.sync_copy(agg_sh.at[rr], rows0)
        pltpu.sync_copy(rows0, agg_out.at[cid, rr])


@functools.partial(
    pl.kernel,
    out_type=jax.ShapeDtypeStruct((E,), f32),
    mesh=_mesh,
    scratch_types=(
        pltpu.VMEM((EPW,), i32), pltpu.VMEM((EPW,), i32),
        pltpu.VMEM((TAILS,), i32), pltpu.VMEM((TAILS,), i32),
        pltpu.VMEM((CS, 128), i32), pltpu.VMEM((CS, 128), i32),
        pltpu.VMEM((CS, 128), i32), pltpu.VMEM((CS, 128), i32),
        pltpu.VMEM((TAILS, 128), i32), pltpu.VMEM((TAILS, 128), i32),
        pltpu.VMEM((CS,), f32), pltpu.VMEM((CS,), f32),
        pltpu.VMEM((TAILS,), f32),
        pltpu.SemaphoreType.DMA, pltpu.SemaphoreType.DMA,
        pltpu.SemaphoreType.DMA, pltpu.SemaphoreType.DMA,
        pltpu.SemaphoreType.DMA, pltpu.SemaphoreType.DMA,
    ),
    compiler_params=pltpu.CompilerParams(needs_layout_passes=False,
                                         disable_bounds_checks=True),
)
def _sc_score(h2_hbm, src_hbm, dst_hbm,
              score_out,
              srcall, dstall, srcv_t, dstv_t,
              rs0, rd0, rs1, rd1, rs_t, rd_t,
              sc0, sc1, sc_t,
              sems0, semd0, sems1, semd1, semw0, semw1):
    cid = lax.axis_index("c")
    sid = lax.axis_index("s")
    w = cid * NS + sid
    base0 = w * EPW
    pltpu.sync_copy(src_hbm.at[pl.ds(base0, EPW)], srcall)
    pltpu.sync_copy(dst_hbm.at[pl.ds(base0, EPW)], dstall)
    rs = (rs0, rs1)
    rd = (rd0, rd1)
    sc = (sc0, sc1)
    sems = (sems0, sems1)
    semd = (semd0, semd1)
    semw = (semw0, semw1)

    zero16 = jnp.zeros((16,), f32)
    lane0 = lax.iota(i32, 16) == 0

    def dot_chunk(rs_ref, rd_ref, sc_ref, n_edges):
        def quad(q, _):
            for u in range(4):
                e = q * 4 + u
                a = [zero16, zero16, zero16, zero16]
                for o in range(8):
                    sl = pl.ds(o * 16, 16)
                    vs = plsc.bitcast(rs_ref[e, sl], jnp.bfloat16)
                    vd = plsc.bitcast(rd_ref[e, sl], jnp.bfloat16)
                    pa, pb = plsc.unpack(
                        vs * vd, format=plsc.PackFormat.INTERLEAVED,
                        preferred_element_type=f32)
                    a[o % 4] = a[o % 4] + pa
                    a[(o + 2) % 4] = a[(o + 2) % 4] + pb
                s = jnp.sum((a[0] + a[1]) + (a[2] + a[3]))
                plsc.store_scatter(sc_ref, [jnp.full((16,), e, dtype=i32)],
                                   jnp.full((16,), s, dtype=f32), mask=lane0)
            return 0

        lax.fori_loop(0, n_edges // 4, quad, 0)

    def fire(j, b):
        pltpu.async_copy(h2_hbm.at[srcall.at[pl.ds(j * CS, CS)]], rs[b],
                         sems[b])
        pltpu.async_copy(h2_hbm.at[dstall.at[pl.ds(j * CS, CS)]], rd[b],
                         semd[b])

    def compute(j, b):
        pltpu.make_async_copy(h2_hbm.at[srcall.at[pl.ds(j * CS, CS)]],
                              rs[b], sems[b]).wait()
        pltpu.make_async_copy(h2_hbm.at[dstall.at[pl.ds(j * CS, CS)]],
                              rd[b], semd[b]).wait()
        pl.when(j >= 2)(
            lambda: pltpu.make_async_copy(
                sc[b], score_out.at[pl.ds(base0, CS)], semw[b]).wait())
        dot_chunk(rs[b], rd[b], sc[b], CS)
        pltpu.async_copy(sc[b], score_out.at[pl.ds(base0 + j * CS, CS)],
                         semw[b])

    fire(0, 0)

    def outer(it, _):
        i0 = it * 2
        fire(i0 + 1, 1)
        compute(i0, 0)
        pl.when(i0 + 2 < NFULLS)(lambda: fire(i0 + 2, 0))
        compute(i0 + 1, 1)
        return 0

    lax.fori_loop(0, NFULLS // 2, outer, 0)
    pltpu.make_async_copy(sc[0], score_out.at[pl.ds(base0, CS)],
                          semw[0]).wait()
    pltpu.make_async_copy(sc[1], score_out.at[pl.ds(base0, CS)],
                          semw[1]).wait()
    baset = base0 + NFULLS * CS
    pltpu.sync_copy(src_hbm.at[pl.ds(baset, TAILS)], srcv_t)
    pltpu.sync_copy(dst_hbm.at[pl.ds(baset, TAILS)], dstv_t)
    cp1 = pltpu.async_copy(h2_hbm.at[srcv_t], rs_t, sems0)
    cp2 = pltpu.async_copy(h2_hbm.at[dstv_t], rd_t, semd0)
    cp1.wait()
    cp2.wait()
    dot_chunk(rs_t, rd_t, sc_t, TAILS)
    pltpu.sync_copy(sc_t, score_out.at[pl.ds(baset, TAILS)])


BN = 1024


def _tc1_body(x_ref, a0_ref, a1_ref, d_ref, ws_ref, wn_ref, b_ref,
              ha_ref, hb_ref):
    deg = jnp.sum(d_ref[...], axis=0)[:, None]
    inv = 1.0 / jnp.maximum(deg, 1.0)
    hn = (a0_ref[...] + a1_ref[...]) * inv
    h = (jnp.dot(x_ref[...], ws_ref[...], preferred_element_type=f32)
         + jnp.dot(hn, wn_ref[...], preferred_element_type=f32)
         + b_ref[...])
    h = jnp.maximum(h, 0.0)
    ha_ref[...] = h[:, :D_IN]
    hb_ref[...] = h[:, D_IN:]


_tc1 = pl.pallas_call(
    _tc1_body,
    grid=(NP // BN,),
    in_specs=[
        pl.BlockSpec((BN, D_IN), lambda i: (i, 0)),
        pl.BlockSpec((BN, D_IN), lambda i: (i, 0)),
        pl.BlockSpec((BN, D_IN), lambda i: (i, 0)),
        pl.BlockSpec((NW, BN), lambda i: (0, i)),
        pl.BlockSpec((D_IN, D_HID), lambda i: (0, 0)),
        pl.BlockSpec((D_IN, D_HID), lambda i: (0, 0)),
        pl.BlockSpec((1, D_HID), lambda i: (0, 0)),
    ],
    out_specs=[pl.BlockSpec((BN, D_IN), lambda i: (i, 0)),
               pl.BlockSpec((BN, D_IN), lambda i: (i, 0))],
    out_shape=[jax.ShapeDtypeStruct((NP, D_IN), f32),
               jax.ShapeDtypeStruct((NP, D_IN), f32)],
)


def _tc2_body(ha_ref, hb_ref, a0_ref, a1_ref, d_ref, ws_ref, wn_ref,
              b_ref, h2_ref):
    deg = jnp.sum(d_ref[...], axis=0)[:, None]
    inv = 1.0 / jnp.maximum(deg, 1.0)
    h1 = jnp.concatenate([ha_ref[...], hb_ref[...]], axis=1)
    hn = jnp.concatenate([a0_ref[...], a1_ref[...]], axis=1) * inv
    h2 = (jnp.dot(h1, ws_ref[...], preferred_element_type=f32)
          + jnp.dot(hn, wn_ref[...], preferred_element_type=f32)
          + b_ref[...])
    h2_ref[...] = jnp.maximum(h2, 0.0).astype(jnp.bfloat16)


_tc2 = pl.pallas_call(
    _tc2_body,
    grid=(NP // BN,),
    in_specs=[
        pl.BlockSpec((BN, D_IN), lambda i: (i, 0)),
        pl.BlockSpec((BN, D_IN), lambda i: (i, 0)),
        pl.BlockSpec((BN, D_IN), lambda i: (i, 0)),
        pl.BlockSpec((BN, D_IN), lambda i: (i, 0)),
        pl.BlockSpec((NW, BN), lambda i: (0, i)),
        pl.BlockSpec((D_HID, D_HID), lambda i: (0, 0)),
        pl.BlockSpec((D_HID, D_HID), lambda i: (0, 0)),
        pl.BlockSpec((1, D_HID), lambda i: (0, 0)),
    ],
    out_specs=pl.BlockSpec((BN, D_HID), lambda i: (i, 0)),
    out_shape=jax.ShapeDtypeStruct((NP, D_HID), jnp.bfloat16),
)


def kernel(x, edge_index, W_self1, W_neigh1, b1, W_self2, W_neigh2, b2):
    src = edge_index[0].astype(i32)
    dst = edge_index[1].astype(i32)
    xp = jnp.pad(x, ((0, NP - N), (0, 0)))
    z = jnp.zeros((CA, D_IN), f32)
    z1d = jnp.zeros((NP,), f32)
    aggp, degp = _sc_agg1(xp, src, dst, z, z1d)
    h1a, h1b = _tc1(xp, aggp[0], aggp[1], degp,
                    W_self1, W_neigh1, b1.reshape(1, -1))
    agg2p = _sc_agg2(h1a, h1b, src, dst, z)
    h2p = _tc2(h1a, h1b, agg2p[0], agg2p[1], degp,
               W_self2, W_neigh2, b2.reshape(1, -1))
    h2w = lax.bitcast_convert_type(h2p.reshape(NP, 128, 2), i32)
    score = _sc_score(h2w, src, dst)
    return score.reshape(E, 1)


# speedup vs baseline: 1.3495x; 1.0002x over previous
import functools

import jax
import jax.numpy as jnp
from jax import lax
from jax.experimental import pallas as pl
from jax.experimental.pallas import tpu as pltpu
from jax.experimental.pallas import tpu_sc as plsc

N = 10000
E = 320000
D_IN = 128
D_HID = 256

NC = 2
NS = 16
NW = NC * NS
NP = 10240
ROWS_W = NP // NS
C = 128
EPW = E // NW
NFULL = EPW // C
TAIL = EPW - NFULL * C
EPS = E // NS
NFULL2 = EPS // C
TAIL2 = EPS - NFULL2 * C
CA = 64
NFA = EPW // CA
TAILA = EPW - NFA * CA
NFC = EPS // CA
TAILC = EPS - NFC * CA
CS = 96
NFULLS = EPW // CS
TAILS = EPW - NFULLS * CS

f32 = jnp.float32
i32 = jnp.int32

_mesh = plsc.VectorSubcoreMesh(core_axis_name="c", subcore_axis_name="s")


@functools.partial(
    pl.kernel,
    out_type=(jax.ShapeDtypeStruct((NC, NP, D_IN), f32),
              jax.ShapeDtypeStruct((NW, NP), f32)),
    mesh=_mesh,
    scratch_types=(
        pltpu.VMEM((EPW,), i32),
        pltpu.VMEM((CA,), i32), pltpu.VMEM((CA,), i32),
        pltpu.VMEM((TAILA,), i32), pltpu.VMEM((TAILA,), i32),
        pltpu.VMEM((CA, D_IN), f32), pltpu.VMEM((CA, D_IN), f32),
        pltpu.VMEM((TAILA, D_IN), f32),
        pltpu.VMEM((NP,), f32),
        pltpu.VMEM_SHARED((NP, D_IN), f32),
        pltpu.SemaphoreType.DMA, pltpu.SemaphoreType.DMA,
        pltpu.SemaphoreType.DMA, pltpu.SemaphoreType.DMA,
    ),
    compiler_params=pltpu.CompilerParams(needs_layout_passes=False,
                                         disable_bounds_checks=True),
)
def _sc_agg1(x_hbm, src_hbm, dst_hbm, z_hbm, z1d_hbm,
             agg_out, deg_out,
             srcall, dstv0, dstv1, srcv_t, dstv_t,
             rows0, rows1, rows_t, hist,
             agg_sh, sem0, sem1, semi0, semi1):
    cid = lax.axis_index("c")
    sid = lax.axis_index("s")
    w = cid * NS + sid
    base0 = w * EPW
    pltpu.sync_copy(src_hbm.at[pl.ds(base0, EPW)], srcall)
    pltpu.sync_copy(z_hbm, rows0)
    pltpu.sync_copy(z1d_hbm, hist)
    for t in range(ROWS_W // CA):
        rr = pl.ds(sid * ROWS_W + t * CA, CA)
        pltpu.sync_copy(rows0, agg_sh.at[rr])
    plsc.subcore_barrier()
    ones16 = jnp.ones((16,), f32)
    dstv = (dstv0, dstv1)
    rows = (rows0, rows1)
    sems = (sem0, sem1)
    semi = (semi0, semi1)

    def count(dref, n):
        for k in range(n // 16):
            idx16 = dref[pl.ds(k * 16, 16)]
            plsc.addupdate_scatter(hist, [idx16], ones16)

    def fire(j, b):
        pltpu.async_copy(dst_hbm.at[pl.ds(base0 + j * CA, CA)], dstv[b],
                         semi[b])
        pltpu.async_copy(x_hbm.at[srcall.at[pl.ds(j * CA, CA)]], rows[b],
                         sems[b])

    def drain_and_scatter(j, b):
        pltpu.make_async_copy(dst_hbm.at[pl.ds(base0 + j * CA, CA)], dstv[b],
                              semi[b]).wait()
        pltpu.make_async_copy(x_hbm.at[srcall.at[pl.ds(j * CA, CA)]], rows[b],
                              sems[b]).wait()
        pltpu.sync_copy(rows[b], agg_sh.at[dstv[b]], add=True)
        count(dstv[b], CA)

    fire(0, 0)

    def outer(it, _):
        i0 = it * 2
        fire(i0 + 1, 1)
        drain_and_scatter(i0, 0)
        pl.when(i0 + 2 < NFA)(lambda: fire(i0 + 2, 0))
        drain_and_scatter(i0 + 1, 1)
        return 0

    lax.fori_loop(0, NFA // 2, outer, 0)
    baset = base0 + NFA * CA
    pltpu.sync_copy(src_hbm.at[pl.ds(baset, TAILA)], srcv_t)
    pltpu.sync_copy(dst_hbm.at[pl.ds(baset, TAILA)], dstv_t)
    pltpu.async_copy(x_hbm.at[srcv_t], rows_t, sem0).wait()
    pltpu.sync_copy(rows_t, agg_sh.at[dstv_t], add=True)
    count(dstv_t, TAILA)
    pltpu.sync_copy(hist, deg_out.at[w])
    plsc.subcore_barrier()
    for t in range(ROWS_W // CA):
        rr = pl.ds(sid * ROWS_W + t * CA, CA)
        pltpu.sync_copy(agg_sh.at[rr], rows0)
        pltpu.sync_copy(rows0, agg_out.at[cid, rr])


@functools.partial(
    pl.kernel,
    out_type=jax.ShapeDtypeStruct((NC, NP, D_IN), f32),
    mesh=_mesh,
    scratch_types=(
        pltpu.VMEM((EPS,), i32),
        pltpu.VMEM((CA,), i32), pltpu.VMEM((CA,), i32),
        pltpu.VMEM((TAILC,), i32), pltpu.VMEM((TAILC,), i32),
        pltpu.VMEM((CA, D_IN), f32), pltpu.VMEM((CA, D_IN), f32),
        pltpu.VMEM((TAILC, D_IN), f32),
        pltpu.VMEM_SHARED((NP, D_IN), f32),
        pltpu.SemaphoreType.DMA, pltpu.SemaphoreType.DMA,
        pltpu.SemaphoreType.DMA, pltpu.SemaphoreType.DMA,
    ),
    compiler_params=pltpu.CompilerParams(needs_layout_passes=False,
                                         disable_bounds_checks=True),
)
def _sc_agg2(h1a_hbm, h1b_hbm, src_hbm, dst_hbm, z_hbm,
             agg_out,
             srcall, dstv0, dstv1, srcv_t, dstv_t,
             rows0, rows1, rows_t,
             agg_sh, sem0, sem1, semi0, semi1):
    cid = lax.axis_index("c")
    sid = lax.axis_index("s")
    base0 = sid * EPS
    pltpu.sync_copy(src_hbm.at[pl.ds(base0, EPS)], srcall)
    pltpu.sync_copy(z_hbm, rows0)
    for t in range(ROWS_W // CA):
        rr = pl.ds(sid * ROWS_W + t * CA, CA)
        pltpu.sync_copy(rows0, agg_sh.at[rr])
    plsc.subcore_barrier()
    dstv = (dstv0, dstv1)
    rows = (rows0, rows1)
    sems = (sem0, sem1)
    semi = (semi0, semi1)

    def fire(j, b):
        pltpu.async_copy(dst_hbm.at[pl.ds(base0 + j * CA, CA)], dstv[b],
                         semi[b])

        @pl.when(cid == 0)
        def _():
            pltpu.async_copy(h1a_hbm.at[srcall.at[pl.ds(j * CA, CA)]],
                             rows[b], sems[b])

        @pl.when(cid == 1)
        def _():
            pltpu.async_copy(h1b_hbm.at[srcall.at[pl.ds(j * CA, CA)]],
                             rows[b], sems[b])

    def drain_and_scatter(j, b):
        pltpu.make_async_copy(dst_hbm.at[pl.ds(base0 + j * CA, CA)], dstv[b],
                              semi[b]).wait()
        pltpu.make_async_copy(h1a_hbm.at[srcall.at[pl.ds(j * CA, CA)]],
                              rows[b], sems[b]).wait()
        pltpu.sync_copy(rows[b], agg_sh.at[dstv[b]], add=True)

    fire(0, 0)

    def outer(it, _):
        i0 = it * 2
        fire(i0 + 1, 1)
        drain_and_scatter(i0, 0)
        pl.when(i0 + 2 < NFC)(lambda: fire(i0 + 2, 0))
        drain_and_scatter(i0 + 1, 1)
        return 0

    lax.fori_loop(0, NFC // 2, outer, 0)
    baset = base0 + NFC * CA
    pltpu.sync_copy(src_hbm.at[pl.ds(baset, TAILC)], srcv_t)
    pltpu.sync_copy(dst_hbm.at[pl.ds(baset, TAILC)], dstv_t)
    @pl.when(cid == 0)
    def _():
        pltpu.async_copy(h1a_hbm.at[srcv_t], rows_t, sem0)

    @pl.when(cid == 1)
    def _():
        pltpu.async_copy(h1b_hbm.at[srcv_t], rows_t, sem0)

    pltpu.make_async_copy(h1a_hbm.at[srcv_t], rows_t, sem0).wait()
    pltpu.sync_copy(rows_t, agg_sh.at[dstv_t], add=True)
    plsc.subcore_barrier()
    for t in range(ROWS_W // CA):
        rr = pl.ds(sid * ROWS_W + t * CA, CA)
        pltpu.sync_copy(agg_sh.at[rr], rows0)
        pltpu.sync_copy(rows0, agg_out.at[cid, rr])


@functools.partial(
    pl.kernel,
    out_type=jax.ShapeDtypeStruct((E,), f32),
    mesh=_mesh,
    scratch_types=(
        pltpu.VMEM((EPW,), i32), pltpu.VMEM((EPW,), i32),
        pltpu.VMEM((TAILS,), i32), pltpu.VMEM((TAILS,), i32),
        pltpu.VMEM((CS, 128), i32), pltpu.VMEM((CS, 128), i32),
        pltpu.VMEM((CS, 128), i32), pltpu.VMEM((CS, 128), i32),
        pltpu.VMEM((TAILS, 128), i32), pltpu.VMEM((TAILS, 128), i32),
        pltpu.VMEM((CS,), f32), pltpu.VMEM((CS,), f32),
        pltpu.VMEM((TAILS,), f32),
        pltpu.SemaphoreType.DMA, pltpu.SemaphoreType.DMA,
        pltpu.SemaphoreType.DMA, pltpu.SemaphoreType.DMA,
        pltpu.SemaphoreType.DMA, pltpu.SemaphoreType.DMA,
    ),
    compiler_params=pltpu.CompilerParams(needs_layout_passes=False,
                                         disable_bounds_checks=True),
)
def _sc_score(h2_hbm, src_hbm, dst_hbm,
              score_out,
              srcall, dstall, srcv_t, dstv_t,
              rs0, rd0, rs1, rd1, rs_t, rd_t,
              sc0, sc1, sc_t,
              sems0, semd0, sems1, semd1, semw0, semw1):
    cid = lax.axis_index("c")
    sid = lax.axis_index("s")
    w = cid * NS + sid
    base0 = w * EPW
    pltpu.sync_copy(src_hbm.at[pl.ds(base0, EPW)], srcall)
    pltpu.sync_copy(dst_hbm.at[pl.ds(base0, EPW)], dstall)
    rs = (rs0, rs1)
    rd = (rd0, rd1)
    sc = (sc0, sc1)
    sems = (sems0, sems1)
    semd = (semd0, semd1)
    semw = (semw0, semw1)

    zero16 = jnp.zeros((16,), f32)
    lane0 = lax.iota(i32, 16) == 0

    def dot_chunk(rs_ref, rd_ref, sc_ref, n_edges):
        def quad(q, _):
            for u in range(8):
                e = q * 8 + u
                a = [zero16, zero16, zero16, zero16]
                for o in range(8):
                    sl = pl.ds(o * 16, 16)
                    vs = plsc.bitcast(rs_ref[e, sl], jnp.bfloat16)
                    vd = plsc.bitcast(rd_ref[e, sl], jnp.bfloat16)
                    pa, pb = plsc.unpack(
                        vs * vd, format=plsc.PackFormat.INTERLEAVED,
                        preferred_element_type=f32)
                    a[o % 4] = a[o % 4] + pa
                    a[(o + 2) % 4] = a[(o + 2) % 4] + pb
                s = jnp.sum((a[0] + a[1]) + (a[2] + a[3]))
                plsc.store_scatter(sc_ref, [jnp.full((16,), e, dtype=i32)],
                                   jnp.full((16,), s, dtype=f32), mask=lane0)
            return 0

        lax.fori_loop(0, n_edges // 8, quad, 0)

    def fire(j, b):
        pltpu.async_copy(h2_hbm.at[srcall.at[pl.ds(j * CS, CS)]], rs[b],
                         sems[b])
        pltpu.async_copy(h2_hbm.at[dstall.at[pl.ds(j * CS, CS)]], rd[b],
                         semd[b])

    def compute(j, b):
        pltpu.make_async_copy(h2_hbm.at[srcall.at[pl.ds(j * CS, CS)]],
                              rs[b], sems[b]).wait()
        pltpu.make_async_copy(h2_hbm.at[dstall.at[pl.ds(j * CS, CS)]],
                              rd[b], semd[b]).wait()
        pl.when(j >= 2)(
            lambda: pltpu.make_async_copy(
                sc[b], score_out.at[pl.ds(base0, CS)], semw[b]).wait())
        dot_chunk(rs[b], rd[b], sc[b], CS)
        pltpu.async_copy(sc[b], score_out.at[pl.ds(base0 + j * CS, CS)],
                         semw[b])

    fire(0, 0)

    def outer(it, _):
        i0 = it * 2
        fire(i0 + 1, 1)
        compute(i0, 0)
        pl.when(i0 + 2 < NFULLS)(lambda: fire(i0 + 2, 0))
        compute(i0 + 1, 1)
        return 0

    lax.fori_loop(0, NFULLS // 2, outer, 0)
    pltpu.make_async_copy(sc[0], score_out.at[pl.ds(base0, CS)],
                          semw[0]).wait()
    pltpu.make_async_copy(sc[1], score_out.at[pl.ds(base0, CS)],
                          semw[1]).wait()
    baset = base0 + NFULLS * CS
    pltpu.sync_copy(src_hbm.at[pl.ds(baset, TAILS)], srcv_t)
    pltpu.sync_copy(dst_hbm.at[pl.ds(baset, TAILS)], dstv_t)
    cp1 = pltpu.async_copy(h2_hbm.at[srcv_t], rs_t, sems0)
    cp2 = pltpu.async_copy(h2_hbm.at[dstv_t], rd_t, semd0)
    cp1.wait()
    cp2.wait()
    dot_chunk(rs_t, rd_t, sc_t, TAILS)
    pltpu.sync_copy(sc_t, score_out.at[pl.ds(baset, TAILS)])


BN = 1024


def _tc1_body(x_ref, a0_ref, a1_ref, d_ref, ws_ref, wn_ref, b_ref,
              ha_ref, hb_ref):
    deg = jnp.sum(d_ref[...], axis=0)[:, None]
    inv = 1.0 / jnp.maximum(deg, 1.0)
    hn = (a0_ref[...] + a1_ref[...]) * inv
    h = (jnp.dot(x_ref[...], ws_ref[...], preferred_element_type=f32)
         + jnp.dot(hn, wn_ref[...], preferred_element_type=f32)
         + b_ref[...])
    h = jnp.maximum(h, 0.0)
    ha_ref[...] = h[:, :D_IN]
    hb_ref[...] = h[:, D_IN:]


_tc1 = pl.pallas_call(
    _tc1_body,
    grid=(NP // BN,),
    in_specs=[
        pl.BlockSpec((BN, D_IN), lambda i: (i, 0)),
        pl.BlockSpec((BN, D_IN), lambda i: (i, 0)),
        pl.BlockSpec((BN, D_IN), lambda i: (i, 0)),
        pl.BlockSpec((NW, BN), lambda i: (0, i)),
        pl.BlockSpec((D_IN, D_HID), lambda i: (0, 0)),
        pl.BlockSpec((D_IN, D_HID), lambda i: (0, 0)),
        pl.BlockSpec((1, D_HID), lambda i: (0, 0)),
    ],
    out_specs=[pl.BlockSpec((BN, D_IN), lambda i: (i, 0)),
               pl.BlockSpec((BN, D_IN), lambda i: (i, 0))],
    out_shape=[jax.ShapeDtypeStruct((NP, D_IN), f32),
               jax.ShapeDtypeStruct((NP, D_IN), f32)],
)


def _tc2_body(ha_ref, hb_ref, a0_ref, a1_ref, d_ref, ws_ref, wn_ref,
              b_ref, h2_ref):
    deg = jnp.sum(d_ref[...], axis=0)[:, None]
    inv = 1.0 / jnp.maximum(deg, 1.0)
    h1 = jnp.concatenate([ha_ref[...], hb_ref[...]], axis=1)
    hn = jnp.concatenate([a0_ref[...], a1_ref[...]], axis=1) * inv
    h2 = (jnp.dot(h1, ws_ref[...], preferred_element_type=f32)
          + jnp.dot(hn, wn_ref[...], preferred_element_type=f32)
          + b_ref[...])
    h2_ref[...] = jnp.maximum(h2, 0.0).astype(jnp.bfloat16)


_tc2 = pl.pallas_call(
    _tc2_body,
    grid=(NP // BN,),
    in_specs=[
        pl.BlockSpec((BN, D_IN), lambda i: (i, 0)),
        pl.BlockSpec((BN, D_IN), lambda i: (i, 0)),
        pl.BlockSpec((BN, D_IN), lambda i: (i, 0)),
        pl.BlockSpec((BN, D_IN), lambda i: (i, 0)),
        pl.BlockSpec((NW, BN), lambda i: (0, i)),
        pl.BlockSpec((D_HID, D_HID), lambda i: (0, 0)),
        pl.BlockSpec((D_HID, D_HID), lambda i: (0, 0)),
        pl.BlockSpec((1, D_HID), lambda i: (0, 0)),
    ],
    out_specs=pl.BlockSpec((BN, D_HID), lambda i: (i, 0)),
    out_shape=jax.ShapeDtypeStruct((NP, D_HID), jnp.bfloat16),
)


def kernel(x, edge_index, W_self1, W_neigh1, b1, W_self2, W_neigh2, b2):
    src = edge_index[0].astype(i32)
    dst = edge_index[1].astype(i32)
    xp = jnp.pad(x, ((0, NP - N), (0, 0)))
    z = jnp.zeros((CA, D_IN), f32)
    z1d = jnp.zeros((NP,), f32)
    aggp, degp = _sc_agg1(xp, src, dst, z, z1d)
    h1a, h1b = _tc1(xp, aggp[0], aggp[1], degp,
                    W_self1, W_neigh1, b1.reshape(1, -1))
    agg2p = _sc_agg2(h1a, h1b, src, dst, z)
    h2p = _tc2(h1a, h1b, agg2p[0], agg2p[1], degp,
               W_self2, W_neigh2, b2.reshape(1, -1))
    h2w = lax.bitcast_convert_type(h2p.reshape(NP, 128, 2), i32)
    score = _sc_score(h2w, src, dst)
    return score.reshape(E, 1)
